# baseline (speedup 1.0000x reference)
import functools

import jax
import jax.numpy as jnp
from jax import lax
from jax.experimental import pallas as pl
from jax.experimental.pallas import tpu as pltpu
from jax.experimental.pallas import tpu_sc as plsc

_B, _L, _H, _D = 8, 4096, 8, 128
_Q = 8
_ROWS = _B * _L
_RD = _H * _D
_NW = 32
_RPW = _ROWS // _NW
_C = 32
_NCHUNK = _RPW // _C
_NB = 2
_BLK = 1024


def _sc_v_body(v_hbm, nv_hbm, len_hbm, nlen_hbm, ov_hbm,
               vbuf, len_v, nlen_v, siv, sov):
    wid = lax.axis_index("s") * 2 + lax.axis_index("c")
    base = wid * _RPW

    pltpu.sync_copy(len_hbm, len_v.at[pl.ds(0, _B)])
    pltpu.sync_copy(nlen_hbm, nlen_v.at[pl.ds(0, _B)])

    def in_v(c, j):
        return pltpu.make_async_copy(
            v_hbm.at[pl.ds(base + c * _C, _C)], vbuf.at[j], siv[j])

    def out_v(c, j):
        return pltpu.make_async_copy(
            vbuf.at[j], ov_hbm.at[pl.ds(base + c * _C, _C)], sov[j])

    for j in range(_NB):
        in_v(j, j).start()

    def step(g, _):
        for j in range(_NB):
            c = _NB * g + j
            in_v(c, j).wait()
            out_v(c, j).start()
            out_v(c, j).wait()

            @pl.when(c + _NB < _NCHUNK)
            def _(c=c, j=j):
                in_v(c + _NB, j).start()
        return _

    lax.fori_loop(0, _NCHUNK // _NB, step, None)

    lv = len_v[...]
    nlv = nlen_v[...]
    for b in range(_B):
        l = lv[b]
        nl = nlv[b]
        for q in range(_Q):
            row = b * _L + l + q
            hit = (q < nl) & (row >= base) & (row < base + _RPW)

            @pl.when(hit)
            def _(row=row, b=b, q=q):
                pltpu.sync_copy(nv_hbm.at[pl.ds(b * _Q + q, 1)],
                                ov_hbm.at[pl.ds(row, 1)])


def _tc_k_body(lengths_ref, new_lengths_ref, k_ref, nk_ref,
               ok_ref, olen_ref):
    b = pl.program_id(0)
    j = pl.program_id(1)
    base = j * _BLK
    l = lengths_ref[b]
    nl = new_lengths_ref[b]

    ok_ref[...] = k_ref[...]

    for q in range(_Q):
        pos = l + q
        hit = (q < nl) & (pos >= base) & (pos < base + _BLK)

        @pl.when(hit)
        def _():
            ok_ref[0, pl.ds(pos - base, 1), :, :] = nk_ref[0, pl.ds(q, 1), :, :]

    @pl.when(j == 0)
    def _():
        olen_ref[b] = l + nl


@jax.jit
def kernel(keys, values, lengths, new_keys, new_values, new_lengths):
    B, L, H, D = keys.shape

    v2 = values.reshape(_ROWS, _RD)
    nv2 = new_values.reshape(_B * _Q, _RD)

    sc_v = pl.kernel(
        _sc_v_body,
        out_type=jax.ShapeDtypeStruct((_ROWS, _RD), values.dtype),
        mesh=plsc.VectorSubcoreMesh(core_axis_name="c", subcore_axis_name="s"),
        scratch_types=[
            pltpu.VMEM((_NB, _C, _RD), jnp.float32),
            pltpu.VMEM((16,), jnp.int32),
            pltpu.VMEM((16,), jnp.int32),
            [pltpu.SemaphoreType.DMA] * _NB,
            [pltpu.SemaphoreType.DMA] * _NB,
        ],
    )
    ov2 = sc_v(v2, nv2, lengths, new_lengths)

    kv_spec = pl.BlockSpec((1, _BLK, H, D), lambda b, j, *_: (b, j, 0, 0))
    new_spec = pl.BlockSpec((1, _Q, H, D), lambda b, j, *_: (b, 0, 0, 0))

    out_k, out_len = pl.pallas_call(
        _tc_k_body,
        grid_spec=pltpu.PrefetchScalarGridSpec(
            num_scalar_prefetch=2,
            grid=(B, L // _BLK),
            in_specs=[kv_spec, new_spec],
            out_specs=[
                kv_spec,
                pl.BlockSpec(memory_space=pltpu.SMEM),
            ],
        ),
        out_shape=[
            jax.ShapeDtypeStruct((B, L, H, D), keys.dtype),
            jax.ShapeDtypeStruct((B,), jnp.int32),
        ],
        compiler_params=pltpu.CompilerParams(
            dimension_semantics=("arbitrary", "arbitrary"),
        ),
    )(lengths, new_lengths, keys, new_keys)

    return (out_k, ov2.reshape(_B, _L, _H, _D), out_len)


# speedup vs baseline: 1.9760x; 1.9760x over previous
import functools

import jax
import jax.numpy as jnp
from jax import lax
from jax.experimental import pallas as pl
from jax.experimental.pallas import tpu as pltpu
from jax.experimental.pallas import tpu_sc as plsc

_B, _L, _H, _D = 8, 4096, 8, 128
_Q = 8
_NW = 32
_TPB = _NW // _B
_RPW = _L // _TPB
_C = 32
_NCHUNK = _RPW // _C
_NB = 2
_BLK = 1024


def _sc_v_body(v_hbm, nv_hbm, len_hbm, nlen_hbm, ov_hbm,
               vbuf, len_v, nlen_v, siv, sov):
    wid = lax.axis_index("s") * 2 + lax.axis_index("c")
    b0 = wid // _TPB
    pos0 = (wid % _TPB) * _RPW

    pltpu.sync_copy(len_hbm, len_v.at[pl.ds(0, _B)])
    pltpu.sync_copy(nlen_hbm, nlen_v.at[pl.ds(0, _B)])

    def in_v(c, j):
        return pltpu.make_async_copy(
            v_hbm.at[b0, pl.ds(pos0 + c * _C, _C)], vbuf.at[j], siv[j])

    def out_v(c, j):
        return pltpu.make_async_copy(
            vbuf.at[j], ov_hbm.at[b0, pl.ds(pos0 + c * _C, _C)], sov[j])

    for j in range(_NB):
        in_v(j, j).start()

    def step(g, _):
        for j in range(_NB):
            c = _NB * g + j
            in_v(c, j).wait()
            out_v(c, j).start()
            out_v(c, j).wait()

            @pl.when(c + _NB < _NCHUNK)
            def _(c=c, j=j):
                in_v(c + _NB, j).start()
        return _

    lax.fori_loop(0, _NCHUNK // _NB, step, None)

    lv = len_v[...]
    nlv = nlen_v[...]
    for b in range(_B):
        l = lv[b]
        nl = nlv[b]
        for q in range(_Q):
            pos = l + q
            hit = (b0 == b) & (q < nl) & (pos >= pos0) & (pos < pos0 + _RPW)

            @pl.when(hit)
            def _(pos=pos, b=b, q=q):
                pltpu.sync_copy(nv_hbm.at[b, pl.ds(q, 1)],
                                ov_hbm.at[b, pl.ds(pos, 1)])


def _tc_k_body(lengths_ref, new_lengths_ref, k_ref, nk_ref,
               ok_ref, olen_ref):
    b = pl.program_id(0)
    j = pl.program_id(1)
    base = j * _BLK
    l = lengths_ref[b]
    nl = new_lengths_ref[b]

    ok_ref[...] = k_ref[...]

    for q in range(_Q):
        pos = l + q
        hit = (q < nl) & (pos >= base) & (pos < base + _BLK)

        @pl.when(hit)
        def _():
            ok_ref[0, pl.ds(pos - base, 1), :, :] = nk_ref[0, pl.ds(q, 1), :, :]

    @pl.when(j == 0)
    def _():
        olen_ref[b] = l + nl


@jax.jit
def kernel(keys, values, lengths, new_keys, new_values, new_lengths):
    B, L, H, D = keys.shape

    sc_v = pl.kernel(
        _sc_v_body,
        out_type=jax.ShapeDtypeStruct((_B, _L, _H, _D), values.dtype),
        mesh=plsc.VectorSubcoreMesh(core_axis_name="c", subcore_axis_name="s"),
        scratch_types=[
            pltpu.VMEM((_NB, _C, _H, _D), jnp.float32),
            pltpu.VMEM((16,), jnp.int32),
            pltpu.VMEM((16,), jnp.int32),
            [pltpu.SemaphoreType.DMA] * _NB,
            [pltpu.SemaphoreType.DMA] * _NB,
        ],
    )
    out_v = sc_v(values, new_values, lengths, new_lengths)

    kv_spec = pl.BlockSpec((1, _BLK, H, D), lambda b, j, *_: (b, j, 0, 0))
    new_spec = pl.BlockSpec((1, _Q, H, D), lambda b, j, *_: (b, 0, 0, 0))

    out_k, out_len = pl.pallas_call(
        _tc_k_body,
        grid_spec=pltpu.PrefetchScalarGridSpec(
            num_scalar_prefetch=2,
            grid=(B, L // _BLK),
            in_specs=[kv_spec, new_spec],
            out_specs=[
                kv_spec,
                pl.BlockSpec(memory_space=pltpu.SMEM),
            ],
        ),
        out_shape=[
            jax.ShapeDtypeStruct((B, L, H, D), keys.dtype),
            jax.ShapeDtypeStruct((B,), jnp.int32),
        ],
        compiler_params=pltpu.CompilerParams(
            dimension_semantics=("arbitrary", "arbitrary"),
        ),
    )(lengths, new_lengths, keys, new_keys)

    return (out_k, out_v, out_len)


# speedup vs baseline: 1.9848x; 1.0045x over previous
import functools

import jax
import jax.numpy as jnp
from jax import lax
from jax.experimental import pallas as pl
from jax.experimental.pallas import tpu as pltpu
from jax.experimental.pallas import tpu_sc as plsc

_B, _L, _H, _D = 8, 4096, 8, 128
_Q = 8
_NW = 32
_TPB = _NW // _B
_RPW = _L // _TPB
_C = 16
_NCHUNK = _RPW // _C
_NB = 4
_LA = 2
_BLK = 1024


def _sc_v_body(v_hbm, nv_hbm, len_hbm, nlen_hbm, ov_hbm,
               vbuf, len_v, nlen_v, siv, sov):
    wid = lax.axis_index("s") * 2 + lax.axis_index("c")
    b0 = wid // _TPB
    pos0 = (wid % _TPB) * _RPW

    pltpu.sync_copy(len_hbm, len_v.at[pl.ds(0, _B)])
    pltpu.sync_copy(nlen_hbm, nlen_v.at[pl.ds(0, _B)])

    def in_v(c, j):
        return pltpu.make_async_copy(
            v_hbm.at[b0, pl.ds(pos0 + c * _C, _C)], vbuf.at[j], siv[j])

    def out_v(c, j):
        return pltpu.make_async_copy(
            vbuf.at[j], ov_hbm.at[b0, pl.ds(pos0 + c * _C, _C)], sov[j])

    for j in range(_LA):
        in_v(j, j).start()

    def step(g, _):
        for j in range(_NB):
            c = _NB * g + j
            in_v(c, j).wait()
            out_v(c, j).start()

            p = c + _LA
            jp = (j + _LA) % _NB

            @pl.when(p < _NCHUNK)
            def _(p=p, jp=jp):
                @pl.when(p >= _NB)
                def _():
                    out_v(p - _NB, jp).wait()
                in_v(p, jp).start()
        return _

    lax.fori_loop(0, _NCHUNK // _NB, step, None)

    for t in range(_NB):
        c = _NCHUNK - _NB + t
        out_v(c, t).wait()

    lv = len_v[...]
    nlv = nlen_v[...]
    for b in range(_B):
        l = lv[b]
        nl = nlv[b]
        for q in range(_Q):
            pos = l + q
            hit = (b0 == b) & (q < nl) & (pos >= pos0) & (pos < pos0 + _RPW)

            @pl.when(hit)
            def _(pos=pos, b=b, q=q):
                pltpu.sync_copy(nv_hbm.at[b, pl.ds(q, 1)],
                                ov_hbm.at[b, pl.ds(pos, 1)])


def _tc_k_body(lengths_ref, new_lengths_ref, k_ref, nk_ref,
               ok_ref, olen_ref):
    b = pl.program_id(0)
    j = pl.program_id(1)
    base = j * _BLK
    l = lengths_ref[b]
    nl = new_lengths_ref[b]

    ok_ref[...] = k_ref[...]

    for q in range(_Q):
        pos = l + q
        hit = (q < nl) & (pos >= base) & (pos < base + _BLK)

        @pl.when(hit)
        def _():
            ok_ref[0, pl.ds(pos - base, 1), :, :] = nk_ref[0, pl.ds(q, 1), :, :]

    @pl.when(j == 0)
    def _():
        olen_ref[b] = l + nl


@jax.jit
def kernel(keys, values, lengths, new_keys, new_values, new_lengths):
    B, L, H, D = keys.shape

    sc_v = pl.kernel(
        _sc_v_body,
        out_type=jax.ShapeDtypeStruct((_B, _L, _H, _D), values.dtype),
        mesh=plsc.VectorSubcoreMesh(core_axis_name="c", subcore_axis_name="s"),
        scratch_types=[
            pltpu.VMEM((_NB, _C, _H, _D), jnp.float32),
            pltpu.VMEM((16,), jnp.int32),
            pltpu.VMEM((16,), jnp.int32),
            [pltpu.SemaphoreType.DMA] * _NB,
            [pltpu.SemaphoreType.DMA] * _NB,
        ],
    )
    out_v = sc_v(values, new_values, lengths, new_lengths)

    kv_spec = pl.BlockSpec((1, _BLK, H, D), lambda b, j, *_: (b, j, 0, 0))
    new_spec = pl.BlockSpec((1, _Q, H, D), lambda b, j, *_: (b, 0, 0, 0))

    out_k, out_len = pl.pallas_call(
        _tc_k_body,
        grid_spec=pltpu.PrefetchScalarGridSpec(
            num_scalar_prefetch=2,
            grid=(B, L // _BLK),
            in_specs=[kv_spec, new_spec],
            out_specs=[
                kv_spec,
                pl.BlockSpec(memory_space=pltpu.SMEM),
            ],
        ),
        out_shape=[
            jax.ShapeDtypeStruct((B, L, H, D), keys.dtype),
            jax.ShapeDtypeStruct((B,), jnp.int32),
        ],
        compiler_params=pltpu.CompilerParams(
            dimension_semantics=("arbitrary", "arbitrary"),
        ),
    )(lengths, new_lengths, keys, new_keys)

    return (out_k, out_v, out_len)
